# baseline (speedup 1.0000x reference)
import functools
import math

import jax
import jax.numpy as jnp
from jax import lax
from jax.experimental import pallas as pl
from jax.experimental.pallas import tpu as pltpu
from jax.experimental.pallas import tpu_sc as plsc

CSIZE = 1024
CDIM = 64
HIDDEN = 768
THETA = 10000.0

TOKENS = 4096
DIST_ROWS = 2 * CSIZE + 1
PER_SUFFIX_ROWS = 2 * (3 * CSIZE + DIST_ROWS)
Y_OFF = 3 * CSIZE + DIST_ROWS
CHUNK = 128
NCHUNK = TOKENS // CHUNK


def _sc_gather(idx_hbm, table_hbm):
    mesh = plsc.VectorSubcoreMesh(core_axis_name="c", subcore_axis_name="s")

    @functools.partial(
        pl.kernel,
        out_type=jax.ShapeDtypeStruct((2, TOKENS, 16, CDIM), jnp.float32),
        mesh=mesh,
        scratch_types=[
            pltpu.VMEM((NCHUNK, CHUNK), jnp.int32),
            pltpu.VMEM((4, CHUNK, CDIM), jnp.float32),
            pltpu.SemaphoreType.DMA((4,)),
        ],
        compiler_params=pltpu.CompilerParams(
            use_tc_tiling_on_sc=False, skip_device_barrier=True),
    )
    def k(idx_ref, table_ref, out_ref, idx_v, buf, gsem):
        sfx = lax.axis_index("c")
        j = lax.axis_index("s")

        xy = j >= 8
        jj = j - jnp.where(xy, 8, 0)
        is_dist = jj >= 3
        off = (sfx * PER_SUFFIX_ROWS
               + jnp.where(xy, Y_OFF, 0)
               + jnp.where(is_dist, 3 * CSIZE + CSIZE, jj * CSIZE))

        pltpu.sync_copy(idx_ref.at[j], idx_v)

        def fix_chunk(r):
            for u in range(CHUNK // 16):
                v = idx_v[r, pl.ds(u * 16, 16)]
                cv = jnp.minimum(jnp.maximum(v, -CSIZE), CSIZE)
                idx_v[r, pl.ds(u * 16, 16)] = jnp.where(is_dist, cv, v) + off

        def fire(c):
            pltpu.async_copy(
                table_ref.at[idx_v.at[c]], buf.at[c % 4], gsem.at[c % 4])

        for c in range(4):
            fix_chunk(c)
            fire(c)

        def do_chunk(c, _):
            b = c % 4
            pltpu.make_async_copy(
                table_ref.at[idx_v.at[c]], buf.at[b], gsem.at[b]).wait()
            pltpu.sync_copy(
                buf.at[b],
                out_ref.at[sfx, pl.ds(c * CHUNK, CHUNK), j],
            )

            @pl.when(c < NCHUNK - 4)
            def _():
                fix_chunk(c + 4)
                fire(c + 4)

            return 0

        lax.fori_loop(0, NCHUNK, do_chunk, 0)

    return k(idx_hbm, table_hbm)


def _rope_body(cos_ref, sin_ref):
    i = pl.program_id(0)
    blk = cos_ref.shape[1]
    pos = (lax.broadcasted_iota(jnp.int32, (blk, HIDDEN // 2), 0)
           + i * blk).astype(jnp.float32)
    half = lax.broadcasted_iota(
        jnp.int32, (blk, HIDDEN // 2), 1).astype(jnp.float32)
    inv_freq = jnp.exp(half * (-2.0 * math.log(THETA) / HIDDEN))
    freqs = pos * inv_freq
    emb = jnp.concatenate([freqs, freqs], axis=-1)
    cos_ref[...] = jnp.broadcast_to(jnp.cos(emb)[None], cos_ref.shape)
    sin_ref[...] = jnp.broadcast_to(jnp.sin(emb)[None], sin_ref.shape)


def _rope(batch, seq):
    blk = 256
    spec = pl.BlockSpec((batch, blk, HIDDEN), lambda i: (0, i, 0))
    shape = jax.ShapeDtypeStruct((batch, seq, HIDDEN), jnp.float32)
    return pl.pallas_call(
        _rope_body,
        grid=(seq // blk,),
        out_specs=[spec, spec],
        out_shape=[shape, shape],
    )()


def kernel(x_features, y_features, x_tl_pos_v, x_br_pos_v, w_pos_v, x_tl_dist_v, y_tl_pos_v, y_br_pos_v, h_pos_v, y_tl_dist_v, x_tl_pos_t, x_br_pos_t, w_pos_t, x_tl_dist_t, y_tl_pos_t, y_br_pos_t, h_pos_t, y_tl_dist_t):
    batch, seq, _ = x_features.shape

    table = jnp.concatenate([
        x_tl_pos_v, x_br_pos_v, w_pos_v, x_tl_dist_v,
        y_tl_pos_v, y_br_pos_v, h_pos_v, y_tl_dist_v,
        x_tl_pos_t, x_br_pos_t, w_pos_t, x_tl_dist_t,
        y_tl_pos_t, y_br_pos_t, h_pos_t, y_tl_dist_t,
    ], axis=0)

    idx = jnp.concatenate([
        x_features.reshape(TOKENS, 8).T,
        y_features.reshape(TOKENS, 8).T,
    ], axis=0).reshape(16, NCHUNK, CHUNK)

    out = _sc_gather(idx, table)
    cos, sin = _rope(batch, seq)
    v_emb = out[0].reshape(batch, seq, 16 * CDIM)
    t_emb = out[1].reshape(batch, seq, 16 * CDIM)
    return v_emb, t_emb, cos, sin


# speedup vs baseline: 1.1135x; 1.1135x over previous
import functools
import math

import jax
import jax.numpy as jnp
from jax import lax
from jax.experimental import pallas as pl
from jax.experimental.pallas import tpu as pltpu
from jax.experimental.pallas import tpu_sc as plsc

CSIZE = 1024
CDIM = 64
HIDDEN = 768
THETA = 10000.0

TOKENS = 4096
DIST_ROWS = 2 * CSIZE + 1
TABLE_ROWS = 2 * (3 * CSIZE + DIST_ROWS)
Y_OFF = 3 * CSIZE + DIST_ROWS
CHUNK = 128
HALF_TOK = TOKENS // 2
NCHUNK = HALF_TOK // CHUNK
NBUF = 4


def _sc_gather(idx_hbm, table_hbm):
    mesh = plsc.VectorSubcoreMesh(core_axis_name="c", subcore_axis_name="s")

    @functools.partial(
        pl.kernel,
        out_type=jax.ShapeDtypeStruct((TOKENS, 32 * CDIM), jnp.float32),
        mesh=mesh,
        scratch_types=[
            pltpu.VMEM((NCHUNK, CHUNK), jnp.int32),
            pltpu.VMEM((NBUF, CHUNK, 2 * CDIM), jnp.float32),
            pltpu.SemaphoreType.DMA((NBUF,)),
        ],
    )
    def k(idx_ref, table_ref, out_ref, idx_v, buf, gsem):
        h = lax.axis_index("c")
        j = lax.axis_index("s")

        xy = j >= 8
        jj = j - jnp.where(xy, 8, 0)
        is_dist = jj >= 3
        off = (jnp.where(xy, Y_OFF, 0)
               + jnp.where(is_dist, 3 * CSIZE + CSIZE, jj * CSIZE))

        pltpu.sync_copy(idx_ref.at[j, h], idx_v)

        def fix_chunk(r):
            for u in range(CHUNK // 16):
                v = idx_v[r, pl.ds(u * 16, 16)]
                cv = jnp.minimum(jnp.maximum(v, -CSIZE), CSIZE)
                idx_v[r, pl.ds(u * 16, 16)] = jnp.where(is_dist, cv, v) + off

        def fire(c):
            pltpu.async_copy(
                table_ref.at[idx_v.at[c]], buf.at[c % NBUF],
                gsem.at[c % NBUF])

        for c in range(NBUF):
            fix_chunk(c)
            fire(c)

        tok0 = h * HALF_TOK

        def do_chunk(c, _):
            b = c % NBUF
            pltpu.make_async_copy(
                table_ref.at[idx_v.at[c]], buf.at[b], gsem.at[b]).wait()
            pltpu.sync_copy(
                buf.at[b],
                out_ref.at[pl.ds(tok0 + c * CHUNK, CHUNK),
                           pl.ds(j * 2 * CDIM, 2 * CDIM)],
            )

            @pl.when(c < NCHUNK - NBUF)
            def _():
                fix_chunk(c + NBUF)
                fire(c + NBUF)

            return 0

        lax.fori_loop(0, NCHUNK, do_chunk, 0)

    return k(idx_hbm, table_hbm)


def _rope_body(cos_ref, sin_ref):
    i = pl.program_id(0)
    blk = cos_ref.shape[1]
    pos = (lax.broadcasted_iota(jnp.int32, (blk, HIDDEN // 2), 0)
           + i * blk).astype(jnp.float32)
    half = lax.broadcasted_iota(
        jnp.int32, (blk, HIDDEN // 2), 1).astype(jnp.float32)
    inv_freq = jnp.exp(half * (-2.0 * math.log(THETA) / HIDDEN))
    freqs = pos * inv_freq
    emb = jnp.concatenate([freqs, freqs], axis=-1)
    cos_ref[...] = jnp.broadcast_to(jnp.cos(emb)[None], cos_ref.shape)
    sin_ref[...] = jnp.broadcast_to(jnp.sin(emb)[None], sin_ref.shape)


def _rope(batch, seq):
    blk = 256
    spec = pl.BlockSpec((batch, blk, HIDDEN), lambda i: (0, i, 0))
    shape = jax.ShapeDtypeStruct((batch, seq, HIDDEN), jnp.float32)
    return pl.pallas_call(
        _rope_body,
        grid=(seq // blk,),
        out_specs=[spec, spec],
        out_shape=[shape, shape],
    )()


def kernel(x_features, y_features, x_tl_pos_v, x_br_pos_v, w_pos_v, x_tl_dist_v, y_tl_pos_v, y_br_pos_v, h_pos_v, y_tl_dist_v, x_tl_pos_t, x_br_pos_t, w_pos_t, x_tl_dist_t, y_tl_pos_t, y_br_pos_t, h_pos_t, y_tl_dist_t):
    batch, seq, _ = x_features.shape

    table_v = jnp.concatenate([
        x_tl_pos_v, x_br_pos_v, w_pos_v, x_tl_dist_v,
        y_tl_pos_v, y_br_pos_v, h_pos_v, y_tl_dist_v,
    ], axis=0)
    table_t = jnp.concatenate([
        x_tl_pos_t, x_br_pos_t, w_pos_t, x_tl_dist_t,
        y_tl_pos_t, y_br_pos_t, h_pos_t, y_tl_dist_t,
    ], axis=0)
    table = jnp.concatenate([table_v, table_t], axis=1)

    idx = jnp.concatenate([
        x_features.reshape(TOKENS, 8).T,
        y_features.reshape(TOKENS, 8).T,
    ], axis=0).reshape(16, 2, NCHUNK, CHUNK)

    out = _sc_gather(idx, table)
    cos, sin = _rope(batch, seq)
    vt = out.reshape(batch, seq, 16, 2, CDIM)
    v_emb = vt[:, :, :, 0, :].reshape(batch, seq, 16 * CDIM)
    t_emb = vt[:, :, :, 1, :].reshape(batch, seq, 16 * CDIM)
    return v_emb, t_emb, cos, sin


# speedup vs baseline: 1.8167x; 1.6316x over previous
import functools
import math

import jax
import jax.numpy as jnp
from jax import lax
from jax.experimental import pallas as pl
from jax.experimental.pallas import tpu as pltpu
from jax.experimental.pallas import tpu_sc as plsc

CSIZE = 1024
CDIM = 64
HIDDEN = 768
THETA = 10000.0

TOKENS = 4096
DIST_ROWS = 2 * CSIZE + 1
TABLE_ROWS = 2 * (3 * CSIZE + DIST_ROWS)
Y_OFF = 3 * CSIZE + DIST_ROWS
CHUNK = 128
HALF_TOK = TOKENS // 2
NCHUNK = HALF_TOK // CHUNK
NBUF = 4


def _sc_gather(idx_hbm, table_hbm):
    mesh = plsc.VectorSubcoreMesh(core_axis_name="c", subcore_axis_name="s")

    @functools.partial(
        pl.kernel,
        out_type=jax.ShapeDtypeStruct((TOKENS, 32 * CDIM), jnp.float32),
        mesh=mesh,
        scratch_types=[
            pltpu.VMEM((NCHUNK, CHUNK), jnp.int32),
            pltpu.VMEM((NBUF, CHUNK, 2 * CDIM), jnp.float32),
            pltpu.SemaphoreType.DMA((NBUF,)),
        ],
    )
    def k(idx_ref, table_ref, out_ref, idx_v, buf, gsem):
        h = lax.axis_index("c")
        j = lax.axis_index("s")

        xy = j >= 8
        jj = j - jnp.where(xy, 8, 0)
        is_dist = jj >= 3
        off = (jnp.where(xy, Y_OFF, 0)
               + jnp.where(is_dist, 3 * CSIZE + CSIZE, jj * CSIZE))

        pltpu.sync_copy(idx_ref.at[j, h], idx_v)

        def fix_chunk(r):
            for u in range(CHUNK // 16):
                v = idx_v[r, pl.ds(u * 16, 16)]
                cv = jnp.minimum(jnp.maximum(v, -CSIZE), CSIZE)
                idx_v[r, pl.ds(u * 16, 16)] = jnp.where(is_dist, cv, v) + off

        def fire(c):
            pltpu.async_copy(
                table_ref.at[idx_v.at[c]], buf.at[c % NBUF],
                gsem.at[c % NBUF])

        for c in range(NBUF):
            fix_chunk(c)
            fire(c)

        tok0 = h * HALF_TOK

        def do_chunk(c, _):
            b = c % NBUF
            pltpu.make_async_copy(
                table_ref.at[idx_v.at[c]], buf.at[b], gsem.at[b]).wait()
            pltpu.sync_copy(
                buf.at[b],
                out_ref.at[pl.ds(tok0 + c * CHUNK, CHUNK),
                           pl.ds(j * 2 * CDIM, 2 * CDIM)],
            )

            @pl.when(c < NCHUNK - NBUF)
            def _():
                fix_chunk(c + NBUF)
                fire(c + NBUF)

            return 0

        lax.fori_loop(0, NCHUNK, do_chunk, 0)

    return k(idx_hbm, table_hbm)


def _finish_body(vt_ref, v_ref, t_ref, cos_ref, sin_ref):
    for j in range(16):
        v_ref[0, :, pl.ds(j * CDIM, CDIM)] = vt_ref[:, pl.ds(j * 2 * CDIM, CDIM)]
        t_ref[0, :, pl.ds(j * CDIM, CDIM)] = vt_ref[:, pl.ds(j * 2 * CDIM + CDIM, CDIM)]

    i = pl.program_id(0)
    blk = cos_ref.shape[1]
    pos = (lax.broadcasted_iota(jnp.int32, (blk, HIDDEN // 2), 0)
           + (i % (2048 // blk)) * blk).astype(jnp.float32)
    half = lax.broadcasted_iota(
        jnp.int32, (blk, HIDDEN // 2), 1).astype(jnp.float32)
    inv_freq = jnp.exp(half * (-2.0 * math.log(THETA) / HIDDEN))
    freqs = pos * inv_freq
    emb = jnp.concatenate([freqs, freqs], axis=-1)
    cos_ref[...] = jnp.cos(emb)[None]
    sin_ref[...] = jnp.sin(emb)[None]


def _finish(vt, batch, seq):
    blk = 256
    nblk = seq // blk
    emb_spec = pl.BlockSpec((1, blk, 16 * CDIM),
                            lambda i: (i // nblk, i % nblk, 0))
    rope_spec = pl.BlockSpec((1, blk, HIDDEN),
                             lambda i: (i // nblk, i % nblk, 0))
    emb_shape = jax.ShapeDtypeStruct((batch, seq, 16 * CDIM), jnp.float32)
    rope_shape = jax.ShapeDtypeStruct((batch, seq, HIDDEN), jnp.float32)
    return pl.pallas_call(
        _finish_body,
        grid=(batch * nblk,),
        in_specs=[pl.BlockSpec((blk, 32 * CDIM), lambda i: (i, 0))],
        out_specs=[emb_spec, emb_spec, rope_spec, rope_spec],
        out_shape=[emb_shape, emb_shape, rope_shape, rope_shape],
    )(vt)


def kernel(x_features, y_features, x_tl_pos_v, x_br_pos_v, w_pos_v, x_tl_dist_v, y_tl_pos_v, y_br_pos_v, h_pos_v, y_tl_dist_v, x_tl_pos_t, x_br_pos_t, w_pos_t, x_tl_dist_t, y_tl_pos_t, y_br_pos_t, h_pos_t, y_tl_dist_t):
    batch, seq, _ = x_features.shape

    table_v = jnp.concatenate([
        x_tl_pos_v, x_br_pos_v, w_pos_v, x_tl_dist_v,
        y_tl_pos_v, y_br_pos_v, h_pos_v, y_tl_dist_v,
    ], axis=0)
    table_t = jnp.concatenate([
        x_tl_pos_t, x_br_pos_t, w_pos_t, x_tl_dist_t,
        y_tl_pos_t, y_br_pos_t, h_pos_t, y_tl_dist_t,
    ], axis=0)
    table = jnp.concatenate([table_v, table_t], axis=1)

    idx = jnp.concatenate([
        x_features.reshape(TOKENS, 8).T,
        y_features.reshape(TOKENS, 8).T,
    ], axis=0).reshape(16, 2, NCHUNK, CHUNK)

    out = _sc_gather(idx, table)
    v_emb, t_emb, cos, sin = _finish(out, batch, seq)
    return v_emb, t_emb, cos, sin


# speedup vs baseline: 1.8221x; 1.0030x over previous
import functools
import math

import jax
import jax.numpy as jnp
from jax import lax
from jax.experimental import pallas as pl
from jax.experimental.pallas import tpu as pltpu
from jax.experimental.pallas import tpu_sc as plsc

CSIZE = 1024
CDIM = 64
HIDDEN = 768
THETA = 10000.0

TOKENS = 4096
DIST_ROWS = 2 * CSIZE + 1
TABLE_ROWS = 2 * (3 * CSIZE + DIST_ROWS)
Y_OFF = 3 * CSIZE + DIST_ROWS
CHUNK = 128
HALF_TOK = TOKENS // 2
NCHUNK = HALF_TOK // CHUNK
NBUF = 6


def _sc_gather(idx_hbm, table_hbm):
    mesh = plsc.VectorSubcoreMesh(core_axis_name="c", subcore_axis_name="s")

    @functools.partial(
        pl.kernel,
        out_type=jax.ShapeDtypeStruct((TOKENS, 32 * CDIM), jnp.float32),
        mesh=mesh,
        scratch_types=[
            pltpu.VMEM((NCHUNK, CHUNK), jnp.int32),
            pltpu.VMEM((NBUF, CHUNK, 2 * CDIM), jnp.float32),
            pltpu.SemaphoreType.DMA((NBUF,)),
        ],
    )
    def k(idx_ref, table_ref, out_ref, idx_v, buf, gsem):
        h = lax.axis_index("c")
        j = lax.axis_index("s")

        xy = j >= 8
        jj = j - jnp.where(xy, 8, 0)
        is_dist = jj >= 3
        off = (jnp.where(xy, Y_OFF, 0)
               + jnp.where(is_dist, 3 * CSIZE + CSIZE, jj * CSIZE))

        pltpu.sync_copy(idx_ref.at[j, h], idx_v)

        def fix_chunk(r):
            for u in range(CHUNK // 16):
                v = idx_v[r, pl.ds(u * 16, 16)]
                cv = jnp.minimum(jnp.maximum(v, -CSIZE), CSIZE)
                idx_v[r, pl.ds(u * 16, 16)] = jnp.where(is_dist, cv, v) + off

        def fire(c):
            pltpu.async_copy(
                table_ref.at[idx_v.at[c]], buf.at[c % NBUF],
                gsem.at[c % NBUF])

        for c in range(NBUF):
            fix_chunk(c)
            fire(c)

        tok0 = h * HALF_TOK

        def do_chunk(c, _):
            b = c % NBUF
            pltpu.make_async_copy(
                table_ref.at[idx_v.at[c]], buf.at[b], gsem.at[b]).wait()
            pltpu.sync_copy(
                buf.at[b],
                out_ref.at[pl.ds(tok0 + c * CHUNK, CHUNK),
                           pl.ds(j * 2 * CDIM, 2 * CDIM)],
            )

            @pl.when(c < NCHUNK - NBUF)
            def _():
                fix_chunk(c + NBUF)
                fire(c + NBUF)

            return 0

        lax.fori_loop(0, NCHUNK, do_chunk, 0)

    return k(idx_hbm, table_hbm)


def _finish_body(vt_ref, v_ref, t_ref, cos_ref, sin_ref):
    for j in range(16):
        v_ref[0, :, pl.ds(j * CDIM, CDIM)] = vt_ref[:, pl.ds(j * 2 * CDIM, CDIM)]
        t_ref[0, :, pl.ds(j * CDIM, CDIM)] = vt_ref[:, pl.ds(j * 2 * CDIM + CDIM, CDIM)]

    i = pl.program_id(0)
    blk = cos_ref.shape[1]
    pos = (lax.broadcasted_iota(jnp.int32, (blk, HIDDEN // 2), 0)
           + (i % (2048 // blk)) * blk).astype(jnp.float32)
    half = lax.broadcasted_iota(
        jnp.int32, (blk, HIDDEN // 2), 1).astype(jnp.float32)
    inv_freq = jnp.exp(half * (-2.0 * math.log(THETA) / HIDDEN))
    freqs = pos * inv_freq
    emb = jnp.concatenate([freqs, freqs], axis=-1)
    cos_ref[...] = jnp.cos(emb)[None]
    sin_ref[...] = jnp.sin(emb)[None]


def _finish(vt, batch, seq):
    blk = 256
    nblk = seq // blk
    emb_spec = pl.BlockSpec((1, blk, 16 * CDIM),
                            lambda i: (i // nblk, i % nblk, 0))
    rope_spec = pl.BlockSpec((1, blk, HIDDEN),
                             lambda i: (i // nblk, i % nblk, 0))
    emb_shape = jax.ShapeDtypeStruct((batch, seq, 16 * CDIM), jnp.float32)
    rope_shape = jax.ShapeDtypeStruct((batch, seq, HIDDEN), jnp.float32)
    return pl.pallas_call(
        _finish_body,
        grid=(batch * nblk,),
        in_specs=[pl.BlockSpec((blk, 32 * CDIM), lambda i: (i, 0))],
        out_specs=[emb_spec, emb_spec, rope_spec, rope_spec],
        out_shape=[emb_shape, emb_shape, rope_shape, rope_shape],
    )(vt)


def kernel(x_features, y_features, x_tl_pos_v, x_br_pos_v, w_pos_v, x_tl_dist_v, y_tl_pos_v, y_br_pos_v, h_pos_v, y_tl_dist_v, x_tl_pos_t, x_br_pos_t, w_pos_t, x_tl_dist_t, y_tl_pos_t, y_br_pos_t, h_pos_t, y_tl_dist_t):
    batch, seq, _ = x_features.shape

    table_v = jnp.concatenate([
        x_tl_pos_v, x_br_pos_v, w_pos_v, x_tl_dist_v,
        y_tl_pos_v, y_br_pos_v, h_pos_v, y_tl_dist_v,
    ], axis=0)
    table_t = jnp.concatenate([
        x_tl_pos_t, x_br_pos_t, w_pos_t, x_tl_dist_t,
        y_tl_pos_t, y_br_pos_t, h_pos_t, y_tl_dist_t,
    ], axis=0)
    table = jnp.concatenate([table_v, table_t], axis=1)

    idx = jnp.concatenate([
        x_features.reshape(TOKENS, 8).T,
        y_features.reshape(TOKENS, 8).T,
    ], axis=0).reshape(16, 2, NCHUNK, CHUNK)

    out = _sc_gather(idx, table)
    v_emb, t_emb, cos, sin = _finish(out, batch, seq)
    return v_emb, t_emb, cos, sin


# speedup vs baseline: 1.9221x; 1.0549x over previous
import functools
import math

import jax
import jax.numpy as jnp
from jax import lax
from jax.experimental import pallas as pl
from jax.experimental.pallas import tpu as pltpu
from jax.experimental.pallas import tpu_sc as plsc

CSIZE = 1024
CDIM = 64
HIDDEN = 768
THETA = 10000.0

TOKENS = 4096
DIST_ROWS = 2 * CSIZE + 1
TABLE_ROWS = 2 * (3 * CSIZE + DIST_ROWS)
Y_OFF = 3 * CSIZE + DIST_ROWS
CHUNK = 128
HALF_TOK = TOKENS // 2
NCHUNK = HALF_TOK // CHUNK
NBUF = 6


def _sc_gather(idx_hbm, table_hbm):
    mesh = plsc.VectorSubcoreMesh(core_axis_name="c", subcore_axis_name="s")

    @functools.partial(
        pl.kernel,
        out_type=jax.ShapeDtypeStruct((TOKENS, 32 * CDIM), jnp.float32),
        mesh=mesh,
        scratch_types=[
            pltpu.VMEM((NCHUNK, CHUNK), jnp.int32),
            pltpu.VMEM((NBUF, CHUNK, 2 * CDIM), jnp.float32),
            pltpu.SemaphoreType.DMA((NBUF,)),
        ],
    )
    def k(idx_ref, table_ref, out_ref, idx_v, buf, gsem):
        h = lax.axis_index("c")
        j = lax.axis_index("s")

        xy = j >= 8
        jj = j - jnp.where(xy, 8, 0)
        is_dist = jj >= 3
        off = (jnp.where(xy, Y_OFF, 0)
               + jnp.where(is_dist, 3 * CSIZE + CSIZE, jj * CSIZE))

        pltpu.sync_copy(idx_ref.at[j, h], idx_v)

        def fix_chunk(r):
            for u in range(CHUNK // 16):
                v = idx_v[r, pl.ds(u * 16, 16)]
                cv = jnp.minimum(jnp.maximum(v, -CSIZE), CSIZE)
                idx_v[r, pl.ds(u * 16, 16)] = jnp.where(is_dist, cv, v) + off

        def fire(c):
            pltpu.async_copy(
                table_ref.at[idx_v.at[c]], buf.at[c % NBUF],
                gsem.at[c % NBUF])

        for c in range(NBUF):
            fix_chunk(c)
            fire(c)

        tok0 = h * HALF_TOK

        def do_chunk(c, _):
            b = c % NBUF
            pltpu.make_async_copy(
                table_ref.at[idx_v.at[c]], buf.at[b], gsem.at[b]).wait()
            pltpu.sync_copy(
                buf.at[b],
                out_ref.at[pl.ds(tok0 + c * CHUNK, CHUNK),
                           pl.ds(j * 2 * CDIM, 2 * CDIM)],
            )

            @pl.when(c < NCHUNK - NBUF)
            def _():
                fix_chunk(c + NBUF)
                fire(c + NBUF)

            return 0

        lax.fori_loop(0, NCHUNK, do_chunk, 0)

    return k(idx_hbm, table_hbm)


def _finish_body(vt_ref, v_ref, t_ref):
    for j in range(16):
        v_ref[0, :, pl.ds(j * CDIM, CDIM)] = vt_ref[:, pl.ds(j * 2 * CDIM, CDIM)]
        t_ref[0, :, pl.ds(j * CDIM, CDIM)] = vt_ref[:, pl.ds(j * 2 * CDIM + CDIM, CDIM)]


def _finish(vt, batch, seq):
    blk = 256
    nblk = seq // blk
    emb_spec = pl.BlockSpec((1, blk, 16 * CDIM),
                            lambda i: (i // nblk, i % nblk, 0))
    emb_shape = jax.ShapeDtypeStruct((batch, seq, 16 * CDIM), jnp.float32)
    return pl.pallas_call(
        _finish_body,
        grid=(batch * nblk,),
        in_specs=[pl.BlockSpec((blk, 32 * CDIM), lambda i: (i, 0))],
        out_specs=[emb_spec, emb_spec],
        out_shape=[emb_shape, emb_shape],
    )(vt)


def _rope_body(cos_ref, sin_ref):
    i = pl.program_id(0)
    blk = cos_ref.shape[1]
    pos = (lax.broadcasted_iota(jnp.int32, (blk, HIDDEN // 2), 0)
           + i * blk).astype(jnp.float32)
    half = lax.broadcasted_iota(
        jnp.int32, (blk, HIDDEN // 2), 1).astype(jnp.float32)
    inv_freq = jnp.exp(half * (-2.0 * math.log(THETA) / HIDDEN))
    freqs = pos * inv_freq
    emb = jnp.concatenate([freqs, freqs], axis=-1)
    cos_ref[...] = jnp.broadcast_to(jnp.cos(emb)[None], cos_ref.shape)
    sin_ref[...] = jnp.broadcast_to(jnp.sin(emb)[None], sin_ref.shape)


def _rope(batch, seq):
    blk = 256
    spec = pl.BlockSpec((batch, blk, HIDDEN), lambda i: (0, i, 0))
    shape = jax.ShapeDtypeStruct((batch, seq, HIDDEN), jnp.float32)
    return pl.pallas_call(
        _rope_body,
        grid=(seq // blk,),
        out_specs=[spec, spec],
        out_shape=[shape, shape],
    )()


def kernel(x_features, y_features, x_tl_pos_v, x_br_pos_v, w_pos_v, x_tl_dist_v, y_tl_pos_v, y_br_pos_v, h_pos_v, y_tl_dist_v, x_tl_pos_t, x_br_pos_t, w_pos_t, x_tl_dist_t, y_tl_pos_t, y_br_pos_t, h_pos_t, y_tl_dist_t):
    batch, seq, _ = x_features.shape

    table_v = jnp.concatenate([
        x_tl_pos_v, x_br_pos_v, w_pos_v, x_tl_dist_v,
        y_tl_pos_v, y_br_pos_v, h_pos_v, y_tl_dist_v,
    ], axis=0)
    table_t = jnp.concatenate([
        x_tl_pos_t, x_br_pos_t, w_pos_t, x_tl_dist_t,
        y_tl_pos_t, y_br_pos_t, h_pos_t, y_tl_dist_t,
    ], axis=0)
    table = jnp.concatenate([table_v, table_t], axis=1)

    idx = jnp.concatenate([
        x_features.reshape(TOKENS, 8).T,
        y_features.reshape(TOKENS, 8).T,
    ], axis=0).reshape(16, 2, NCHUNK, CHUNK)

    out = _sc_gather(idx, table)
    cos, sin = _rope(batch, seq)
    v_emb, t_emb = _finish(out, batch, seq)
    return v_emb, t_emb, cos, sin


# speedup vs baseline: 2.3830x; 1.2398x over previous
import functools
import math

import jax
import jax.numpy as jnp
from jax import lax
from jax.experimental import pallas as pl
from jax.experimental.pallas import tpu as pltpu
from jax.experimental.pallas import tpu_sc as plsc

CSIZE = 1024
CDIM = 64
HIDDEN = 768
THETA = 10000.0

TOKENS = 4096
DIST_ROWS = 2 * CSIZE + 1
Y_OFF = 5128
TABLE_ROWS = 10256
CHUNK = 128
HALF_TOK = TOKENS // 2
NCHUNK = HALF_TOK // CHUNK
NBUF = 6


def _sc_gather(idx_hbm, table_hbm):
    mesh = plsc.VectorSubcoreMesh(core_axis_name="c", subcore_axis_name="s")

    @functools.partial(
        pl.kernel,
        out_type=jax.ShapeDtypeStruct((TOKENS, 32 * CDIM), jnp.float32),
        mesh=mesh,
        scratch_types=[
            pltpu.VMEM((NCHUNK, CHUNK), jnp.int32),
            pltpu.VMEM((NBUF, CHUNK, 2 * CDIM), jnp.float32),
            pltpu.SemaphoreType.DMA((NBUF,)),
        ],
    )
    def k(idx_ref, table_ref, out_ref, idx_v, buf, gsem):
        h = lax.axis_index("c")
        j = lax.axis_index("s")

        xy = j >= 8
        jj = j - jnp.where(xy, 8, 0)
        is_dist = jj >= 3
        off = (jnp.where(xy, Y_OFF, 0)
               + jnp.where(is_dist, 3 * CSIZE + CSIZE, jj * CSIZE))

        pltpu.sync_copy(idx_ref.at[j, h], idx_v)

        def fix_chunk(r):
            for u in range(CHUNK // 16):
                v = idx_v[r, pl.ds(u * 16, 16)]
                cv = jnp.minimum(jnp.maximum(v, -CSIZE), CSIZE)
                idx_v[r, pl.ds(u * 16, 16)] = jnp.where(is_dist, cv, v) + off

        def fire(c):
            pltpu.async_copy(
                table_ref.at[idx_v.at[c]], buf.at[c % NBUF],
                gsem.at[c % NBUF])

        for c in range(NBUF):
            fix_chunk(c)
            fire(c)

        tok0 = h * HALF_TOK

        def do_chunk(c, _):
            b = c % NBUF
            pltpu.make_async_copy(
                table_ref.at[idx_v.at[c]], buf.at[b], gsem.at[b]).wait()
            pltpu.sync_copy(
                buf.at[b],
                out_ref.at[pl.ds(tok0 + c * CHUNK, CHUNK),
                           pl.ds(j * 2 * CDIM, 2 * CDIM)],
            )

            @pl.when(c < NCHUNK - NBUF)
            def _():
                fix_chunk(c + NBUF)
                fire(c + NBUF)

            return 0

        lax.fori_loop(0, NCHUNK, do_chunk, 0)

    return k(idx_hbm, table_hbm)


def _finish_body(vt_ref, v_ref, t_ref):
    for j in range(16):
        v_ref[0, :, pl.ds(j * CDIM, CDIM)] = vt_ref[:, pl.ds(j * 2 * CDIM, CDIM)]
        t_ref[0, :, pl.ds(j * CDIM, CDIM)] = vt_ref[:, pl.ds(j * 2 * CDIM + CDIM, CDIM)]


def _finish(vt, batch, seq):
    blk = 256
    nblk = seq // blk
    emb_spec = pl.BlockSpec((1, blk, 16 * CDIM),
                            lambda i: (i // nblk, i % nblk, 0))
    emb_shape = jax.ShapeDtypeStruct((batch, seq, 16 * CDIM), jnp.float32)
    return pl.pallas_call(
        _finish_body,
        grid=(batch * nblk,),
        in_specs=[pl.BlockSpec((blk, 32 * CDIM), lambda i: (i, 0))],
        out_specs=[emb_spec, emb_spec],
        out_shape=[emb_shape, emb_shape],
    )(vt)


def _rope_body(cos_ref, sin_ref):
    i = pl.program_id(0)
    blk = cos_ref.shape[1]
    pos = (lax.broadcasted_iota(jnp.int32, (blk, HIDDEN // 2), 0)
           + i * blk).astype(jnp.float32)
    half = lax.broadcasted_iota(
        jnp.int32, (blk, HIDDEN // 2), 1).astype(jnp.float32)
    inv_freq = jnp.exp(half * (-2.0 * math.log(THETA) / HIDDEN))
    freqs = pos * inv_freq
    emb = jnp.concatenate([freqs, freqs], axis=-1)
    cos_ref[...] = jnp.broadcast_to(jnp.cos(emb)[None], cos_ref.shape)
    sin_ref[...] = jnp.broadcast_to(jnp.sin(emb)[None], sin_ref.shape)


def _rope(batch, seq):
    blk = 256
    spec = pl.BlockSpec((batch, blk, HIDDEN), lambda i: (0, i, 0))
    shape = jax.ShapeDtypeStruct((batch, seq, HIDDEN), jnp.float32)
    return pl.pallas_call(
        _rope_body,
        grid=(seq // blk,),
        out_specs=[spec, spec],
        out_shape=[shape, shape],
    )()


_TAB_OFFS = (0, CSIZE, 2 * CSIZE, 3 * CSIZE,
             Y_OFF, Y_OFF + CSIZE, Y_OFF + 2 * CSIZE, Y_OFF + 3 * CSIZE)


def _prep_body(*refs):
    out_ref = refs[16]
    for k in range(8):
        n = refs[k].shape[0]
        out_ref[pl.ds(_TAB_OFFS[k], n), pl.ds(0, CDIM)] = refs[k][...]
        out_ref[pl.ds(_TAB_OFFS[k], n), pl.ds(CDIM, CDIM)] = refs[k + 8][...]


def _prep_table(*tabs):
    return pl.pallas_call(
        _prep_body,
        out_shape=jax.ShapeDtypeStruct((TABLE_ROWS, 2 * CDIM), jnp.float32),
    )(*tabs)


def kernel(x_features, y_features, x_tl_pos_v, x_br_pos_v, w_pos_v, x_tl_dist_v, y_tl_pos_v, y_br_pos_v, h_pos_v, y_tl_dist_v, x_tl_pos_t, x_br_pos_t, w_pos_t, x_tl_dist_t, y_tl_pos_t, y_br_pos_t, h_pos_t, y_tl_dist_t):
    batch, seq, _ = x_features.shape

    table = _prep_table(
        x_tl_pos_v, x_br_pos_v, w_pos_v, x_tl_dist_v,
        y_tl_pos_v, y_br_pos_v, h_pos_v, y_tl_dist_v,
        x_tl_pos_t, x_br_pos_t, w_pos_t, x_tl_dist_t,
        y_tl_pos_t, y_br_pos_t, h_pos_t, y_tl_dist_t,
    )

    idx = jnp.concatenate([
        x_features.reshape(TOKENS, 8).T,
        y_features.reshape(TOKENS, 8).T,
    ], axis=0).reshape(16, 2, NCHUNK, CHUNK)

    out = _sc_gather(idx, table)
    cos, sin = _rope(batch, seq)
    v_emb, t_emb = _finish(out, batch, seq)
    return v_emb, t_emb, cos, sin
